# Initial kernel scaffold; baseline (speedup 1.0000x reference)
import jax
import jax.numpy as jnp
from jax.experimental import pallas as pl


def kernel(states, edges_mat, priority, edge_index, We, be, Wm, bm, Wu, bu, Wd, bd, Wt, bt):
    raise NotImplementedError("write your pallas kernel here")


# speedup vs baseline: 44.0281x; 44.0281x over previous
import jax
import jax.numpy as jnp
from jax.experimental import pallas as pl

_NEG = -1.0e30
_NP = 256
_H = 32


def _mpnn_body(emw_ref, s0_ref, pT_ref, wehT_ref, we0_ref,
               wms_ref, wmd_ref, bm_ref,
               wuz_ref, wuu_ref, bu_ref,
               wd1_ref, wd2_ref, bd_ref,
               wt_ref, bt_ref, lmask_ref,
               outs_ref, outp_ref):
    f32 = jnp.float32
    dn = (((0,), (0,)), ((), ()))

    lmask = lmask_ref[...]
    pT = pT_ref[...]
    wehT = wehT_ref[...]
    we0 = we0_ref[...]
    wms = wms_ref[...]
    wmd = wmd_ref[...]
    bm_c = bm_ref[...]
    wuz = wuz_ref[...]
    wuu = wuu_ref[...]
    bu_c = bu_ref[...]
    wd1 = wd1_ref[...]
    wd2 = wd2_ref[...]
    bd = bd_ref[0:1, 0:1]
    wt = wt_ref[...]
    bt = bt_ref[0:1, 0:1]

    s0 = s0_ref[...] * lmask
    outs_ref[0:1, :, :] = s0.reshape(1, 1, _NP)
    outp_ref[0:1, :, :] = jnp.zeros((1, 1, 128), f32)

    def step(t, carry):
        state_row, hidden = carry
        zT = jax.lax.dot_general(wehT, hidden, dn, preferred_element_type=f32)
        zT = zT + we0 * state_row + pT
        a_std = jax.lax.dot_general(zT, wms, dn, preferred_element_type=f32)
        bT = jax.lax.dot_general(wmd, zT, dn, preferred_element_type=f32)
        rows = []
        for h in range(_H):
            acol = a_std[:, h:h + 1]
            rows.append(jnp.max(acol + emw_ref[h], axis=0, keepdims=True))
        uT = jnp.concatenate(rows, axis=0) + bT + bm_c
        nh = (jax.lax.dot_general(wuz, zT, dn, preferred_element_type=f32)
              + jax.lax.dot_general(wuu, uT, dn, preferred_element_type=f32)
              + bu_c)
        nh = nh * lmask
        mean_col = jnp.sum(nh, axis=1, keepdims=True) * (1.0 / 255.0)
        sv = jnp.sum(mean_col * wt, axis=0, keepdims=True)
        stop = jax.nn.sigmoid(sv + bt)
        srow = (jax.lax.dot_general(wd1, nh, dn, preferred_element_type=f32)
                + jax.lax.dot_general(wd2, zT, dn, preferred_element_type=f32)
                + bd)
        srow = srow * lmask
        outs_ref[pl.ds(t, 1), :, :] = srow.reshape(1, 1, _NP)
        outp_ref[pl.ds(t, 1), :, :] = jnp.broadcast_to(stop, (1, 128)).reshape(1, 1, 128)
        return (srow, nh)

    jax.lax.fori_loop(1, _NP, step, (s0, jnp.zeros((_H, _NP), f32)))


def kernel(states, edges_mat, priority, edge_index, We, be, Wm, bm, Wu, bu, Wd, bd, Wt, bt):
    f32 = jnp.float32
    T, N = states.shape
    H = _H

    src = edge_index[0].astype(jnp.int32)
    dst = edge_index[1].astype(jnp.int32)

    mask = jnp.zeros((_NP, _NP), jnp.bool_).at[src, dst].set(True)
    em_pad = jnp.zeros((_NP, _NP), f32).at[:N, :N].set(edges_mat)
    w = Wm[2 * H]
    emw = jnp.where(mask[None], em_pad[None] * w[:, None, None],
                    jnp.float32(_NEG))

    s0p = jnp.zeros((1, _NP), f32).at[0, :N].set(states[0])
    prio_p = jnp.zeros((_NP,), f32).at[:N].set(priority)
    pT = We[1 + H][:, None] * prio_p[None, :] + be[:, None]
    lmask = (jnp.arange(_NP) < N).astype(f32)[None, :]

    args = (
        emw, s0p, pT,
        We[1:1 + H],
        We[0][:, None],
        Wm[0:H],
        Wm[H:2 * H],
        bm[:, None],
        Wu[0:H],
        Wu[H:2 * H],
        bu[:, None],
        Wd[0:H],
        Wd[H:2 * H],
        jnp.broadcast_to(bd[None, :], (1, 128)),
        Wt,
        jnp.broadcast_to(bt[None, :], (1, 128)),
        lmask,
    )

    outs, outp = pl.pallas_call(
        _mpnn_body,
        out_shape=(jax.ShapeDtypeStruct((_NP, 1, _NP), f32),
                   jax.ShapeDtypeStruct((_NP, 1, 128), f32)),
    )(*args)

    outs2 = outs.reshape(_NP, _NP)
    preds = outs2.T[:N, :]
    preds_stop = outp.reshape(_NP, 128)[:, 0:1][None, :, :]
    return preds, preds_stop


# speedup vs baseline: 55.4859x; 1.2602x over previous
import jax
import jax.numpy as jnp
from jax.experimental import pallas as pl

_NEG = -1.0e30
_NP = 256
_H = 32


def _mpnn_body(t4_ref, s0_ref, pT_ref, weh_ref, we0_ref,
               wms_ref,
               wuz2_ref, wuu_ref, bu2_ref,
               wd1_ref, wd2_ref, bd_ref,
               wt_ref, bt_ref, lmask_ref,
               outs_ref, outp_ref):
    f32 = jnp.float32
    dn00 = (((0,), (0,)), ((), ()))
    dn01 = (((0,), (1,)), ((), ()))

    lmask = lmask_ref[...]
    pT = pT_ref[...]
    weh = weh_ref[...]
    we0 = we0_ref[...]
    wms = wms_ref[...]
    wuz2 = wuz2_ref[...]
    wuu = wuu_ref[...]
    bu2 = bu2_ref[...]
    wd1 = wd1_ref[...]
    wd2 = wd2_ref[...]
    bd = bd_ref[0:1, 0:1]
    wt = wt_ref[...]
    bt = bt_ref[0:1, 0:1]

    s0 = s0_ref[...] * lmask
    outs_ref[0:1, :, :] = s0.reshape(1, 1, _NP)
    outp_ref[0:1, :, :] = jnp.zeros((1, 1, 128), f32)

    def step(t, carry):
        state_row, hidden = carry
        zT = jax.lax.dot_general(weh, hidden, dn00, preferred_element_type=f32)
        zT = zT + we0 * state_row + pT
        a_std = jax.lax.dot_general(zT, wms, dn00, preferred_element_type=f32)
        a_rep = jnp.concatenate([a_std, a_std, a_std, a_std], axis=1)
        rows = []
        for jj in range(64):
            rows.append(jnp.max(t4_ref[jj] + a_rep, axis=0, keepdims=True))
        u_pack = jnp.concatenate(rows, axis=0)
        u_std = jnp.concatenate(
            [u_pack[:, 0:32], u_pack[:, 32:64],
             u_pack[:, 64:96], u_pack[:, 96:128]], axis=0)
        nh = (jax.lax.dot_general(wuz2, zT, dn00, preferred_element_type=f32)
              + jax.lax.dot_general(wuu, u_std, dn01, preferred_element_type=f32)
              + bu2)
        nh = nh * lmask
        mean_col = jnp.sum(nh, axis=1, keepdims=True) * (1.0 / 255.0)
        sv = jnp.sum(mean_col * wt, axis=0, keepdims=True)
        stop = jax.nn.sigmoid(sv + bt)
        srow = (jax.lax.dot_general(wd1, nh, dn00, preferred_element_type=f32)
                + jax.lax.dot_general(wd2, zT, dn00, preferred_element_type=f32)
                + bd)
        srow = srow * lmask
        outs_ref[pl.ds(t, 1), :, :] = srow.reshape(1, 1, _NP)
        outp_ref[pl.ds(t, 1), :, :] = jnp.broadcast_to(stop, (1, 128)).reshape(1, 1, 128)
        return (srow, nh)

    jax.lax.fori_loop(1, _NP, step, (s0, jnp.zeros((_H, _NP), f32)))


def kernel(states, edges_mat, priority, edge_index, We, be, Wm, bm, Wu, bu, Wd, bd, Wt, bt):
    f32 = jnp.float32
    T, N = states.shape
    H = _H

    src = edge_index[0].astype(jnp.int32)
    dst = edge_index[1].astype(jnp.int32)

    mask = jnp.zeros((_NP, _NP), jnp.bool_).at[src, dst].set(True)
    em_pad = jnp.zeros((_NP, _NP), f32).at[:N, :N].set(edges_mat)
    w = Wm[2 * H]
    emw = jnp.where(mask[:, :, None], em_pad[:, :, None] * w[None, None, :],
                    jnp.float32(_NEG))
    t4 = (emw.transpose(1, 0, 2).reshape(4, 64, _NP, H)
          .transpose(1, 2, 0, 3).reshape(64, _NP, 128))

    s0p = jnp.zeros((1, _NP), f32).at[0, :N].set(states[0])
    prio_p = jnp.zeros((_NP,), f32).at[:N].set(priority)
    pT = We[1 + H][:, None] * prio_p[None, :] + be[:, None]
    lmask = (jnp.arange(_NP) < N).astype(f32)[None, :]

    wuu = Wu[H:2 * H]
    wuz2 = Wu[0:H] + Wm[H:2 * H] @ wuu
    bu2 = (bu + bm @ wuu)[:, None]

    args = (
        t4, s0p, pT,
        We[1:1 + H],
        We[0][:, None],
        Wm[0:H],
        wuz2, wuu, bu2,
        Wd[0:H],
        Wd[H:2 * H],
        jnp.broadcast_to(bd[None, :], (1, 128)),
        Wt,
        jnp.broadcast_to(bt[None, :], (1, 128)),
        lmask,
    )

    outs, outp = pl.pallas_call(
        _mpnn_body,
        out_shape=(jax.ShapeDtypeStruct((_NP, 1, _NP), f32),
                   jax.ShapeDtypeStruct((_NP, 1, 128), f32)),
    )(*args)

    outs2 = outs.reshape(_NP, _NP)
    preds = outs2.T[:N, :]
    preds_stop = outp.reshape(_NP, 128)[:, 0:1][None, :, :]
    return preds, preds_stop


# speedup vs baseline: 65.3020x; 1.1769x over previous
import jax
import jax.numpy as jnp
from jax.experimental import pallas as pl

_NEG = -1.0e30
_NP = 256
_H = 32


def _mpnn_body(t4_ref, s0_ref, z1_ref,
               wms4_ref, wuz2_ref, wuu_ref, bu2_ref,
               g1_ref, g2_ref, c0_ref, maskb_ref,
               wd1_ref, wd2_ref, bd_ref,
               wt_ref, bt_ref,
               outs_ref, outp_ref):
    f32 = jnp.float32
    dn10 = (((1,), (0,)), ((), ()))
    dn01 = (((0,), (1,)), ((), ()))

    wms4 = wms4_ref[...]
    wuz2 = wuz2_ref[...]
    wuu = wuu_ref[...]
    bu2 = bu2_ref[...]
    g1 = g1_ref[...]
    g2 = g2_ref[...]
    c0 = c0_ref[...]
    maskb = maskb_ref[...]
    wd1 = wd1_ref[...]
    wd2 = wd2_ref[...]
    bd = bd_ref[0:1, 0:1]
    wt = wt_ref[...]
    bt = bt_ref[0:1, 0:1]

    outs_ref[0:1, :, :] = s0_ref[...].reshape(1, 1, _NP)
    outp_ref[0:1, :, :] = jnp.zeros((1, 1, 128), f32)

    def step(t, z):
        a_rep = jax.lax.dot_general(z, wms4, dn10, preferred_element_type=f32)
        parts = []
        for jj in range(64):
            x = (t4_ref[jj] + a_rep).reshape(32, 8, 128)
            parts.append(jnp.max(x, axis=0))
        p = jnp.concatenate(parts, axis=0)
        u_pack = jnp.max(p.reshape(64, 8, 128), axis=1)
        u_std = jnp.concatenate(
            [u_pack[:, 0:32], u_pack[:, 32:64],
             u_pack[:, 64:96], u_pack[:, 96:128]], axis=0)
        nh = (jax.lax.dot_general(z, wuz2, dn10, preferred_element_type=f32)
              + jax.lax.dot_general(u_std, wuu, dn10, preferred_element_type=f32)
              + bu2)
        nh = nh * maskb
        mean_row = jnp.sum(nh, axis=0, keepdims=True) * (1.0 / 255.0)
        sv = jax.lax.dot_general(mean_row, wt, dn10, preferred_element_type=f32)
        stop = jax.nn.sigmoid(sv + bt)
        srow = (jax.lax.dot_general(wd1, nh, dn01, preferred_element_type=f32)
                + jax.lax.dot_general(wd2, z, dn01, preferred_element_type=f32)
                + bd)
        outs_ref[pl.ds(t, 1), :, :] = srow.reshape(1, 1, _NP)
        outp_ref[pl.ds(t, 1), :, :] = jnp.broadcast_to(stop, (1, 128)).reshape(1, 1, 128)
        return (jax.lax.dot_general(nh, g1, dn10, preferred_element_type=f32)
                + jax.lax.dot_general(z, g2, dn10, preferred_element_type=f32)
                + c0)

    jax.lax.fori_loop(1, _NP, step, z1_ref[...])


def kernel(states, edges_mat, priority, edge_index, We, be, Wm, bm, Wu, bu, Wd, bd, Wt, bt):
    f32 = jnp.float32
    T, N = states.shape
    H = _H

    src = edge_index[0].astype(jnp.int32)
    dst = edge_index[1].astype(jnp.int32)

    mask = jnp.zeros((_NP, _NP), jnp.bool_).at[src, dst].set(True)
    em_pad = jnp.zeros((_NP, _NP), f32).at[:N, :N].set(edges_mat)
    w = Wm[2 * H]
    emw = jnp.where(mask[:, :, None], em_pad[:, :, None] * w[None, None, :],
                    jnp.float32(_NEG))
    t4 = (emw.transpose(1, 0, 2).reshape(4, 64, _NP, H)
          .transpose(1, 2, 0, 3).reshape(64, _NP, 128))

    s0p = jnp.zeros((1, _NP), f32).at[0, :N].set(states[0])
    prio_p = jnp.zeros((_NP,), f32).at[:N].set(priority)
    we0_row = We[0][None, :]
    p_std = prio_p[:, None] * We[1 + H][None, :] + be[None, :]
    z1 = jnp.zeros((_NP, 1), f32).at[:N, 0].set(states[0]) @ we0_row + p_std

    wuu = Wu[H:2 * H]
    wuz2 = Wu[0:H] + Wm[H:2 * H] @ wuu
    bu2 = (bu + bm @ wuu)[None, :]
    wd1 = Wd[0:H]
    wd2 = Wd[H:2 * H]
    g1 = We[1:1 + H] + wd1 @ we0_row
    g2 = wd2 @ we0_row
    c0 = p_std + bd[0] * jnp.broadcast_to(we0_row, (_NP, H))
    maskb = jnp.broadcast_to((jnp.arange(_NP) < N)[:, None], (_NP, H)).astype(f32)

    args = (
        t4, s0p, z1,
        jnp.concatenate([Wm[0:H]] * 4, axis=1),
        wuz2, wuu, bu2,
        g1, g2, c0, maskb,
        wd1, wd2,
        jnp.broadcast_to(bd[None, :], (1, 128)),
        Wt,
        jnp.broadcast_to(bt[None, :], (1, 128)),
    )

    outs, outp = pl.pallas_call(
        _mpnn_body,
        out_shape=(jax.ShapeDtypeStruct((_NP, 1, _NP), f32),
                   jax.ShapeDtypeStruct((_NP, 1, 128), f32)),
    )(*args)

    outs2 = outs.reshape(_NP, _NP)
    preds = outs2.T[:N, :]
    preds_stop = outp.reshape(_NP, 128)[:, 0:1][None, :, :]
    return preds, preds_stop


# speedup vs baseline: 66.2855x; 1.0151x over previous
import jax
import jax.numpy as jnp
from jax.experimental import pallas as pl

_NEG = -1.0e30
_NP = 256
_H = 32


def _mpnn_body(t4_ref, s0_ref, z1_ref,
               wms4_ref, wuz2_ref, wuu_ref, bu2_ref,
               g1_ref, g2_ref, c0_ref, maskb_ref,
               wd1_ref, wd2_ref, bd_ref,
               wt_ref, bt_ref,
               outs_ref, outp_ref):
    f32 = jnp.float32
    dn10 = (((1,), (0,)), ((), ()))
    dn01 = (((0,), (1,)), ((), ()))

    wms4 = wms4_ref[...]
    wuz2 = wuz2_ref[...]
    wuu = wuu_ref[...]
    bu2 = bu2_ref[...]
    g1 = g1_ref[...]
    g2 = g2_ref[...]
    c0 = c0_ref[...]
    maskb = maskb_ref[...]
    wd1 = wd1_ref[...]
    wd2 = wd2_ref[...]
    bd = bd_ref[0:1, 0:1]
    wt = wt_ref[...]
    bt = bt_ref[0:1, 0:1]

    outs_ref[0:1, :, :] = s0_ref[...].reshape(1, 1, _NP)
    outp_ref[0:1, :, :] = jnp.zeros((1, 1, 128), f32)

    def step(t, z):
        a_rep = jax.lax.dot_general(z, wms4, dn10, preferred_element_type=f32)
        parts = [None] * 64
        for jjt in range(8):
            accs = [None] * 8
            for ci in range(8):
                ar = a_rep[32 * ci:32 * (ci + 1), :]
                for q in range(8):
                    x = (t4_ref[jjt * 8 + q, 32 * ci:32 * (ci + 1), :]
                         + ar).reshape(4, 8, 128)
                    m = jnp.max(x, axis=0)
                    accs[q] = m if ci == 0 else jnp.maximum(accs[q], m)
            for q in range(8):
                parts[jjt * 8 + q] = accs[q]
        p = jnp.concatenate(parts, axis=0)
        u_pack = jnp.max(p.reshape(64, 8, 128), axis=1)
        u_std = jnp.concatenate(
            [u_pack[:, 0:32], u_pack[:, 32:64],
             u_pack[:, 64:96], u_pack[:, 96:128]], axis=0)
        nh = (jax.lax.dot_general(z, wuz2, dn10, preferred_element_type=f32)
              + jax.lax.dot_general(u_std, wuu, dn10, preferred_element_type=f32)
              + bu2)
        nh = nh * maskb
        mean_row = jnp.sum(nh, axis=0, keepdims=True) * (1.0 / 255.0)
        sv = jax.lax.dot_general(mean_row, wt, dn10, preferred_element_type=f32)
        stop = jax.nn.sigmoid(sv + bt)
        srow = (jax.lax.dot_general(wd1, nh, dn01, preferred_element_type=f32)
                + jax.lax.dot_general(wd2, z, dn01, preferred_element_type=f32)
                + bd)
        outs_ref[pl.ds(t, 1), :, :] = srow.reshape(1, 1, _NP)
        outp_ref[pl.ds(t, 1), :, :] = jnp.broadcast_to(stop, (1, 128)).reshape(1, 1, 128)
        return (jax.lax.dot_general(nh, g1, dn10, preferred_element_type=f32)
                + jax.lax.dot_general(z, g2, dn10, preferred_element_type=f32)
                + c0)

    jax.lax.fori_loop(1, _NP, step, z1_ref[...])


def kernel(states, edges_mat, priority, edge_index, We, be, Wm, bm, Wu, bu, Wd, bd, Wt, bt):
    f32 = jnp.float32
    T, N = states.shape
    H = _H

    src = edge_index[0].astype(jnp.int32)
    dst = edge_index[1].astype(jnp.int32)

    mask = jnp.zeros((_NP, _NP), jnp.bool_).at[src, dst].set(True)
    em_pad = jnp.zeros((_NP, _NP), f32).at[:N, :N].set(edges_mat)
    w = Wm[2 * H]
    emw = jnp.where(mask[:, :, None], em_pad[:, :, None] * w[None, None, :],
                    jnp.float32(_NEG))
    t4 = (emw.transpose(1, 0, 2).reshape(4, 64, _NP, H)
          .transpose(1, 2, 0, 3).reshape(64, _NP, 128))

    s0p = jnp.zeros((1, _NP), f32).at[0, :N].set(states[0])
    prio_p = jnp.zeros((_NP,), f32).at[:N].set(priority)
    we0_row = We[0][None, :]
    p_std = prio_p[:, None] * We[1 + H][None, :] + be[None, :]
    z1 = jnp.zeros((_NP, 1), f32).at[:N, 0].set(states[0]) @ we0_row + p_std

    wuu = Wu[H:2 * H]
    wuz2 = Wu[0:H] + Wm[H:2 * H] @ wuu
    bu2 = (bu + bm @ wuu)[None, :]
    wd1 = Wd[0:H]
    wd2 = Wd[H:2 * H]
    g1 = We[1:1 + H] + wd1 @ we0_row
    g2 = wd2 @ we0_row
    c0 = p_std + bd[0] * jnp.broadcast_to(we0_row, (_NP, H))
    maskb = jnp.broadcast_to((jnp.arange(_NP) < N)[:, None], (_NP, H)).astype(f32)

    args = (
        t4, s0p, z1,
        jnp.concatenate([Wm[0:H]] * 4, axis=1),
        wuz2, wuu, bu2,
        g1, g2, c0, maskb,
        wd1, wd2,
        jnp.broadcast_to(bd[None, :], (1, 128)),
        Wt,
        jnp.broadcast_to(bt[None, :], (1, 128)),
    )

    outs, outp = pl.pallas_call(
        _mpnn_body,
        out_shape=(jax.ShapeDtypeStruct((_NP, 1, _NP), f32),
                   jax.ShapeDtypeStruct((_NP, 1, 128), f32)),
    )(*args)

    outs2 = outs.reshape(_NP, _NP)
    preds = outs2.T[:N, :]
    preds_stop = outp.reshape(_NP, 128)[:, 0:1][None, :, :]
    return preds, preds_stop


# speedup vs baseline: 69.8263x; 1.0534x over previous
import jax
import jax.numpy as jnp
from jax.experimental import pallas as pl

_NEG = -1.0e30
_NP = 256
_H = 32


def _mpnn_body(t4_ref, s0_ref, z1_ref,
               wms4_ref, wuz2_ref, wuu_ref, bu2_ref,
               g1_ref, g2_ref, c0_ref, maskb_ref,
               k1_ref, k2_ref, kc_ref,
               wd1_ref, wd2_ref, bd_ref,
               wt_ref, bt_ref,
               outs_ref, outp_ref):
    f32 = jnp.float32
    dn10 = (((1,), (0,)), ((), ()))
    dn01 = (((0,), (1,)), ((), ()))

    wms4 = wms4_ref[...]
    wuz2 = wuz2_ref[...]
    wuu = wuu_ref[...]
    bu2 = bu2_ref[...]
    g1 = g1_ref[...]
    g2 = g2_ref[...]
    c0 = c0_ref[...]
    maskb = maskb_ref[...]
    k1 = k1_ref[...]
    k2 = k2_ref[...]
    kc = kc_ref[...]
    wd1 = wd1_ref[...]
    wd2 = wd2_ref[...]
    bd = bd_ref[0:1, 0:1]
    wt = wt_ref[...]
    bt = bt_ref[0:1, 0:1]

    outs_ref[0:1, :, :] = s0_ref[...].reshape(1, 1, _NP)
    outp_ref[0:1, :, :] = jnp.zeros((1, 1, 128), f32)

    def step(t, carry):
        z, a_rep = carry
        parts = [None] * 64
        for jjt in range(8):
            accs = [None] * 8
            for ci in range(8):
                ar = a_rep[32 * ci:32 * (ci + 1), :]
                for q in range(8):
                    x = (t4_ref[jjt * 8 + q, 32 * ci:32 * (ci + 1), :]
                         + ar).reshape(4, 8, 128)
                    m = jnp.max(x, axis=0)
                    accs[q] = m if ci == 0 else jnp.maximum(accs[q], m)
            for q in range(8):
                parts[jjt * 8 + q] = accs[q]
        p = jnp.concatenate(parts, axis=0)
        u_pack = jnp.max(p.reshape(64, 8, 128), axis=1)
        u_std = jnp.concatenate(
            [u_pack[:, 0:32], u_pack[:, 32:64],
             u_pack[:, 64:96], u_pack[:, 96:128]], axis=0)
        nh = (jax.lax.dot_general(z, wuz2, dn10, preferred_element_type=f32)
              + jax.lax.dot_general(u_std, wuu, dn10, preferred_element_type=f32)
              + bu2)
        nh = nh * maskb
        mean_row = jnp.sum(nh, axis=0, keepdims=True) * (1.0 / 255.0)
        sv = jax.lax.dot_general(mean_row, wt, dn10, preferred_element_type=f32)
        stop = jax.nn.sigmoid(sv + bt)
        srow = (jax.lax.dot_general(wd1, nh, dn01, preferred_element_type=f32)
                + jax.lax.dot_general(wd2, z, dn01, preferred_element_type=f32)
                + bd)
        outs_ref[pl.ds(t, 1), :, :] = srow.reshape(1, 1, _NP)
        outp_ref[pl.ds(t, 1), :, :] = jnp.broadcast_to(stop, (1, 128)).reshape(1, 1, 128)
        z_next = (jax.lax.dot_general(nh, g1, dn10, preferred_element_type=f32)
                  + jax.lax.dot_general(z, g2, dn10, preferred_element_type=f32)
                  + c0)
        a_next = (jax.lax.dot_general(z, k1, dn10, preferred_element_type=f32)
                  + jax.lax.dot_general(u_std, k2, dn10, preferred_element_type=f32)
                  + kc)
        return (z_next, a_next)

    z1 = z1_ref[...]
    a1 = jax.lax.dot_general(z1, wms4, dn10, preferred_element_type=f32)

    def step3(k, carry):
        t = 1 + 3 * k
        carry = step(t, carry)
        carry = step(t + 1, carry)
        return step(t + 2, carry)

    jax.lax.fori_loop(0, 85, step3, (z1, a1))


def kernel(states, edges_mat, priority, edge_index, We, be, Wm, bm, Wu, bu, Wd, bd, Wt, bt):
    f32 = jnp.float32
    T, N = states.shape
    H = _H

    src = edge_index[0].astype(jnp.int32)
    dst = edge_index[1].astype(jnp.int32)

    mask = jnp.zeros((_NP, _NP), jnp.bool_).at[src, dst].set(True)
    em_pad = jnp.zeros((_NP, _NP), f32).at[:N, :N].set(edges_mat)
    w = Wm[2 * H]
    emw = jnp.where(mask[:, :, None], em_pad[:, :, None] * w[None, None, :],
                    jnp.float32(_NEG))
    t4 = (emw.transpose(1, 0, 2).reshape(4, 64, _NP, H)
          .transpose(1, 2, 0, 3).reshape(64, _NP, 128))

    s0p = jnp.zeros((1, _NP), f32).at[0, :N].set(states[0])
    prio_p = jnp.zeros((_NP,), f32).at[:N].set(priority)
    we0_row = We[0][None, :]
    p_std = prio_p[:, None] * We[1 + H][None, :] + be[None, :]
    z1 = jnp.zeros((_NP, 1), f32).at[:N, 0].set(states[0]) @ we0_row + p_std

    wuu = Wu[H:2 * H]
    wuz2 = Wu[0:H] + Wm[H:2 * H] @ wuu
    bu2 = (bu + bm @ wuu)[None, :]
    wd1 = Wd[0:H]
    wd2 = Wd[H:2 * H]
    g1 = We[1:1 + H] + wd1 @ we0_row
    g2 = wd2 @ we0_row
    c0 = p_std + bd[0] * jnp.broadcast_to(we0_row, (_NP, H))
    maskb = jnp.broadcast_to((jnp.arange(_NP) < N)[:, None], (_NP, H)).astype(f32)
    wms4 = jnp.concatenate([Wm[0:H]] * 4, axis=1)
    k1 = (wuz2 @ g1 + g2) @ wms4
    k2 = wuu @ g1 @ wms4
    kc = c0 @ wms4 + jnp.broadcast_to(bu2 @ g1 @ wms4, (_NP, 128))

    args = (
        t4, s0p, z1,
        wms4,
        wuz2, wuu, bu2,
        g1, g2, c0, maskb,
        k1, k2, kc,
        wd1, wd2,
        jnp.broadcast_to(bd[None, :], (1, 128)),
        Wt,
        jnp.broadcast_to(bt[None, :], (1, 128)),
    )

    outs, outp = pl.pallas_call(
        _mpnn_body,
        out_shape=(jax.ShapeDtypeStruct((_NP, 1, _NP), f32),
                   jax.ShapeDtypeStruct((_NP, 1, 128), f32)),
    )(*args)

    outs2 = outs.reshape(_NP, _NP)
    preds = outs2.T[:N, :]
    preds_stop = outp.reshape(_NP, 128)[:, 0:1][None, :, :]
    return preds, preds_stop
